# Initial kernel scaffold; baseline (speedup 1.0000x reference)
import jax
import jax.numpy as jnp
from jax.experimental import pallas as pl


def kernel(features, labels, feature_bank, label_bank):
    raise NotImplementedError("write your pallas kernel here")


# speedup vs baseline: 2.9681x; 2.9681x over previous
import jax
import jax.numpy as jnp
from jax.experimental import pallas as pl

_BANK = 100000
_DIM = 128
_BATCH = 16384
_BLK = 2000
_NB = _BANK // _BLK
_FEAT_LAST = _BATCH // _BLK


def _body(feat_ref, bank_ref, lab_ref, lbank_ref, out_fb_ref, out_lb_ref):
    i = pl.program_id(0)
    base = i * _BLK
    rows2d = base + jax.lax.broadcasted_iota(jnp.int32, (_BLK, _DIM), 0)
    out_fb_ref[...] = jnp.where(rows2d < _BATCH, feat_ref[...], bank_ref[...])

    @pl.when(i == 0)
    def _():
        out_lb_ref[0:_BATCH] = lab_ref[...]
        out_lb_ref[_BATCH:_BANK] = lbank_ref[_BATCH:_BANK]


def kernel(features, labels, feature_bank, label_bank):
    out_fb, out_lb = pl.pallas_call(
        _body,
        grid=(_NB,),
        in_specs=[
            pl.BlockSpec((_BLK, _DIM), lambda i: (jnp.minimum(i, _FEAT_LAST), 0)),
            pl.BlockSpec((_BLK, _DIM), lambda i: (jnp.maximum(i, _FEAT_LAST), 0)),
            pl.BlockSpec((_BATCH,), lambda i: (0,)),
            pl.BlockSpec((_BANK,), lambda i: (0,)),
        ],
        out_specs=[
            pl.BlockSpec((_BLK, _DIM), lambda i: (i, 0)),
            pl.BlockSpec((_BANK,), lambda i: (0,)),
        ],
        out_shape=[
            jax.ShapeDtypeStruct((_BANK, _DIM), feature_bank.dtype),
            jax.ShapeDtypeStruct((_BANK,), label_bank.dtype),
        ],
    )(features, feature_bank, labels, label_bank)
    return out_fb, out_lb
